# baseline (speedup 1.0000x reference)
import jax
import jax.numpy as jnp
from jax.experimental import pallas as pl
from jax.experimental.pallas import tpu as pltpu

_BM = 200
_NBUF = 5


def _fused_kernel(
    x_ref,
    w_ref,
    b_ref,
    adj_ref,
    out_ref,
    h_ref,
    buf_ref,
    sem_ref,
    obuf_ref,
    osem_ref,
    xv_ref,
    xsem_ref,
):
    n = x_ref.shape[0]
    nblocks = n // _BM
    rounds = nblocks // _NBUF

    def copy_in(i, s):
        return pltpu.make_async_copy(
            adj_ref.at[pl.ds(i * _BM, _BM), :],
            buf_ref.at[s],
            sem_ref.at[s],
        )

    def copy_out(i, s):
        return pltpu.make_async_copy(
            obuf_ref.at[s],
            out_ref.at[pl.ds(i * _BM, _BM), :],
            osem_ref.at[s],
        )

    for s in range(_NBUF):
        copy_in(s, s).start()
    x_copy = pltpu.make_async_copy(x_ref, xv_ref, xsem_ref)
    x_copy.start()
    x_copy.wait()

    h_ref[:, :] = (
        jnp.dot(xv_ref[:, :], w_ref[:, :].T, preferred_element_type=jnp.float32)
        + b_ref[:, :]
    )

    def round_body(r, carry):
        for s in range(_NBUF):
            i = r * _NBUF + s
            copy_in(i, s).wait()

            @pl.when(r > 0)
            def _():
                copy_out(i - _NBUF, s).wait()

            acc = jnp.dot(buf_ref[s], h_ref[:, :], preferred_element_type=jnp.float32)
            obuf_ref[s] = jnp.maximum(acc, 0.0)

            @pl.when(r + 1 < rounds)
            def _():
                copy_in(i + _NBUF, s).start()

            copy_out(i, s).start()

        return carry

    jax.lax.fori_loop(0, rounds, round_body, 0)

    for s in range(_NBUF):
        copy_out((rounds - 1) * _NBUF + s, s).wait()


@jax.jit
def kernel(x, adj, W, b):
    n, d_in = x.shape
    d_out = W.shape[0]

    support = pl.pallas_call(
        _fused_kernel,
        in_specs=[
            pl.BlockSpec(memory_space=pl.ANY),
            pl.BlockSpec(memory_space=pltpu.MemorySpace.VMEM),
            pl.BlockSpec(memory_space=pltpu.MemorySpace.VMEM),
            pl.BlockSpec(memory_space=pl.ANY),
        ],
        out_specs=pl.BlockSpec(memory_space=pl.ANY),
        out_shape=jax.ShapeDtypeStruct((n, d_out), jnp.float32),
        scratch_shapes=[
            pltpu.VMEM((n, d_out), jnp.float32),
            pltpu.VMEM((_NBUF, _BM, n), jnp.float32),
            pltpu.SemaphoreType.DMA((_NBUF,)),
            pltpu.VMEM((_NBUF, _BM, d_out), jnp.float32),
            pltpu.SemaphoreType.DMA((_NBUF,)),
            pltpu.VMEM((n, d_in), jnp.float32),
            pltpu.SemaphoreType.DMA,
        ],
    )(x, W, b.reshape(1, d_out), adj)

    return (support, adj)


# speedup vs baseline: 1.0030x; 1.0030x over previous
import jax
import jax.numpy as jnp
from jax.experimental import pallas as pl
from jax.experimental.pallas import tpu as pltpu

_BM = 200
_NBUF = 5


def _fused_kernel(x_ref, w_ref, b_ref, adj_ref, out_ref, h_ref, buf_ref, sem_ref):
    n = x_ref.shape[0]
    nblocks = n // _BM
    rounds = nblocks // _NBUF

    def copy_in(i, s):
        return pltpu.make_async_copy(
            adj_ref.at[pl.ds(i * _BM, _BM), :],
            buf_ref.at[s],
            sem_ref.at[s],
        )

    for s in range(_NBUF):
        copy_in(s, s).start()

    h_ref[:, :] = (
        jnp.dot(x_ref[:, :], w_ref[:, :].T, preferred_element_type=jnp.float32)
        + b_ref[:, :]
    )

    def round_body(r, carry):
        for s in range(_NBUF):
            i = r * _NBUF + s
            copy_in(i, s).wait()

            acc = jnp.dot(buf_ref[s], h_ref[:, :], preferred_element_type=jnp.float32)
            out_ref[pl.ds(i * _BM, _BM), :] = jnp.maximum(acc, 0.0)

            @pl.when(r + 1 < rounds)
            def _():
                copy_in(i + _NBUF, s).start()

        return carry

    jax.lax.fori_loop(0, rounds, round_body, 0)


@jax.jit
def kernel(x, adj, W, b):
    n, d_in = x.shape
    d_out = W.shape[0]

    support = pl.pallas_call(
        _fused_kernel,
        in_specs=[
            pl.BlockSpec(memory_space=pltpu.MemorySpace.VMEM),
            pl.BlockSpec(memory_space=pltpu.MemorySpace.VMEM),
            pl.BlockSpec(memory_space=pltpu.MemorySpace.VMEM),
            pl.BlockSpec(memory_space=pl.ANY),
        ],
        out_specs=pl.BlockSpec(memory_space=pltpu.MemorySpace.VMEM),
        out_shape=jax.ShapeDtypeStruct((n, d_out), jnp.float32),
        scratch_shapes=[
            pltpu.VMEM((n, d_out), jnp.float32),
            pltpu.VMEM((_NBUF, _BM, n), jnp.float32),
            pltpu.SemaphoreType.DMA((_NBUF,)),
        ],
    )(x, W, b.reshape(1, d_out), adj)

    return (support, adj)


# speedup vs baseline: 1.0175x; 1.0144x over previous
import jax
import jax.numpy as jnp
from jax.experimental import pallas as pl
from jax.experimental.pallas import tpu as pltpu

_BM = 80
_NBUF = 5


def _fused_kernel(
    x_ref, w_ref, b_ref, adj_ref, out_ref, h_ref, buf_ref, sem_ref, obuf_ref, osem_ref
):
    n = x_ref.shape[0]
    nblocks = n // _BM
    rounds = nblocks // _NBUF

    def copy_in(i, s):
        return pltpu.make_async_copy(
            adj_ref.at[pl.ds(i * _BM, _BM), :],
            buf_ref.at[s],
            sem_ref.at[s],
        )

    def copy_out(i, s):
        return pltpu.make_async_copy(
            obuf_ref.at[s],
            out_ref.at[pl.ds(i * _BM, _BM), :],
            osem_ref.at[s],
        )

    for s in range(_NBUF):
        copy_in(s, s).start()

    h_ref[:, :] = (
        jnp.dot(x_ref[:, :], w_ref[:, :].T, preferred_element_type=jnp.float32)
        + b_ref[:, :]
    )

    def round_body(r, carry):
        for s in range(_NBUF):
            i = r * _NBUF + s
            copy_in(i, s).wait()

            @pl.when(r > 0)
            def _():
                copy_out(i - _NBUF, s).wait()

            acc = jnp.dot(buf_ref[s], h_ref[:, :], preferred_element_type=jnp.float32)
            obuf_ref[s] = jnp.maximum(acc, 0.0)

            @pl.when(r + 1 < rounds)
            def _():
                copy_in(i + _NBUF, s).start()

            copy_out(i, s).start()

        return carry

    jax.lax.fori_loop(0, rounds, round_body, 0)

    for s in range(_NBUF):
        copy_out((rounds - 1) * _NBUF + s, s).wait()


@jax.jit
def kernel(x, adj, W, b):
    n, d_in = x.shape
    d_out = W.shape[0]

    support = pl.pallas_call(
        _fused_kernel,
        in_specs=[
            pl.BlockSpec(memory_space=pltpu.MemorySpace.VMEM),
            pl.BlockSpec(memory_space=pltpu.MemorySpace.VMEM),
            pl.BlockSpec(memory_space=pltpu.MemorySpace.VMEM),
            pl.BlockSpec(memory_space=pl.ANY),
        ],
        out_specs=pl.BlockSpec(memory_space=pl.ANY),
        out_shape=jax.ShapeDtypeStruct((n, d_out), jnp.float32),
        scratch_shapes=[
            pltpu.VMEM((n, d_out), jnp.float32),
            pltpu.VMEM((_NBUF, _BM, n), jnp.float32),
            pltpu.SemaphoreType.DMA((_NBUF,)),
            pltpu.VMEM((_NBUF, _BM, d_out), jnp.float32),
            pltpu.SemaphoreType.DMA((_NBUF,)),
        ],
    )(x, W, b.reshape(1, d_out), adj)

    return (support, adj)
